# baseline (speedup 1.0000x reference)
import functools

import jax
import jax.numpy as jnp
from jax import lax
from jax.experimental import pallas as pl
from jax.experimental.pallas import tpu as pltpu
from jax.experimental.pallas import tpu_sc as plsc

_info = plsc.get_sparse_core_info()
_NC = _info.num_cores
_NS = _info.num_subcores
_NW = _NC * _NS


def _make_sc_broadcast(B, S, D, dtype):
    CH = 32
    rows_per_w = S // _NW
    n_steps = rows_per_w // CH
    mesh = plsc.VectorSubcoreMesh(core_axis_name="c", subcore_axis_name="s")

    @functools.partial(
        pl.kernel,
        mesh=mesh,
        out_type=jax.ShapeDtypeStruct((B, S, D), dtype),
        scratch_types=[
            pltpu.VMEM((2, CH, D), dtype),
            pltpu.SemaphoreType.DMA,
            pltpu.SemaphoreType.DMA,
        ],
    )
    def sc_kernel(table_hbm, out_hbm, buf, rsem, wsem):
        w = lax.axis_index("s") * _NC + lax.axis_index("c")
        base = w * rows_per_w
        read = pltpu.async_copy(table_hbm.at[pl.ds(base, CH)], buf.at[0], rsem)
        writes = []
        for i in range(n_steps):
            cur = i % 2
            read.wait()
            for h in writes:
                h.wait()
            writes = []
            if i + 1 < n_steps:
                read = pltpu.async_copy(
                    table_hbm.at[pl.ds(base + (i + 1) * CH, CH)],
                    buf.at[(i + 1) % 2],
                    rsem,
                )
            r0 = base + i * CH
            for b in range(B):
                writes.append(
                    pltpu.async_copy(
                        buf.at[cur], out_hbm.at[b, pl.ds(r0, CH)], wsem
                    )
                )
        for h in writes:
            h.wait()

    return sc_kernel


def kernel(inputs, table):
    B = inputs.shape[0]
    S, D = table.shape
    return _make_sc_broadcast(B, S, D, table.dtype)(table)
